# baseline (speedup 1.0000x reference)
import functools

import numpy as np

import jax
import jax.numpy as jnp
from jax import lax
from jax.experimental import pallas as pl
from jax.experimental.pallas import tpu as pltpu
from jax.experimental.pallas import tpu_sc as plsc

VOCAB = 150528
D_MODEL = 1024
BATCH = 4
SEQ = 2048
MASK_TOKEN = 150001

_INFO = plsc.get_sparse_core_info()
_NW = _INFO.num_cores * _INFO.num_subcores
_ROWS = BATCH * SEQ
_RPW = _ROWS // _NW
_CHUNK = 32
_NCH = _RPW // _CHUNK

_mesh = plsc.VectorSubcoreMesh(core_axis_name="c", subcore_axis_name="s")


_NPIECE = 1
_PROWS = _ROWS // _NPIECE
_PNCH = _PROWS // _NW // _CHUNK


def _make_sc_gather(nch):
    @functools.partial(
        pl.kernel,
        mesh=_mesh,
        out_type=jax.ShapeDtypeStruct((SEQ // _NPIECE * 8, D_MODEL),
                                      jnp.float32),
        scratch_types=[
            pltpu.VMEM((nch, _CHUNK), jnp.int32),
            pltpu.VMEM((nch, _CHUNK), jnp.int32),
            pltpu.VMEM((_CHUNK, D_MODEL), jnp.float32),
            pltpu.VMEM((_CHUNK, D_MODEL), jnp.float32),
            pltpu.SemaphoreType.DMA,
            pltpu.SemaphoreType.DMA,
            pltpu.SemaphoreType.DMA,
            pltpu.SemaphoreType.DMA,
        ],
    )
    def _sc_gather(idx_hbm, oidx_hbm, w_hbm, out_hbm, idx_v, oidx_v, buf0,
                   buf1, si0, si1, so0, so1):
        wid = lax.axis_index("s") * _INFO.num_cores + lax.axis_index("c")
        pltpu.sync_copy(idx_hbm.at[wid], idx_v)
        pltpu.sync_copy(oidx_hbm.at[wid], oidx_v)
        bufs = (buf0, buf1)
        sin = (si0, si1)
        sout = (so0, so1)
        cin = [None] * nch
        cout = [None] * nch
        cin[0] = pltpu.async_copy(w_hbm.at[idx_v.at[0]], buf0, si0)
        if nch > 1:
            cin[1] = pltpu.async_copy(w_hbm.at[idx_v.at[1]], buf1, si1)
        for c in range(nch):
            b = c % 2
            cin[c].wait()
            cout[c] = pltpu.async_copy(bufs[b], out_hbm.at[oidx_v.at[c]],
                                       sout[b])
            nxt = c + 2
            if nxt < nch:
                cout[c].wait()
                cin[nxt] = pltpu.async_copy(w_hbm.at[idx_v.at[nxt]], bufs[b],
                                            sin[b])
            else:
                cout[c].wait()

    return _sc_gather


_sc_gather_piece = _make_sc_gather(_PNCH)


_MROWS_PW = _ROWS // _NW
_MHALF = 64
_NHALF = _MROWS_PW // _MHALF
_NCHK = SEQ // 64
_WROW = SEQ // 4
_ONESW = 0x01010101


@functools.partial(
    pl.kernel,
    mesh=_mesh,
    out_type=[
        jax.ShapeDtypeStruct((_ROWS * SEQ // 4,), jnp.int32),
        jax.ShapeDtypeStruct((_ROWS,), jnp.int32),
    ],
    scratch_types=[
        pltpu.VMEM((SEQ,), jnp.int32),
        pltpu.VMEM((_MHALF * _WROW,), jnp.int32),
        pltpu.VMEM((_MHALF * _WROW,), jnp.int32),
        pltpu.VMEM((_MROWS_PW,), jnp.int32),
        pltpu.VMEM((16,), jnp.int32),
        pltpu.SemaphoreType.DMA,
        pltpu.SemaphoreType.DMA,
        pltpu.SemaphoreType.DMA,
    ],
)
def _sc_mask(ids_hbm, mask_hbm, pos_hbm, ids_v, rb0, rb1, pos_v, acc_v, sm0,
             sm1, sp):
    wid = lax.axis_index("s") * _INFO.num_cores + lax.axis_index("c")
    row0 = wid * _MROWS_PW
    b = row0 // SEQ
    i0 = row0 - b * SEQ
    pltpu.sync_copy(ids_hbm.at[b], ids_v)

    acc_v[...] = jnp.full((16,), SEQ, jnp.int32)

    @pl.loop(0, SEQ // 16)
    def _mp_scan(ch):
        v = ids_v[pl.ds(ch * 16, 16)]
        cand = jnp.where(v == MASK_TOKEN,
                         lax.iota(jnp.int32, 16) + ch * 16, SEQ)
        acc_v[...] = jnp.minimum(acc_v[...], cand)

    accs = acc_v[...]
    mp = accs[0]
    for _k in range(1, 16):
        mp = jnp.minimum(mp, accs[_k])

    @pl.loop(0, _MROWS_PW // 16)
    def _pos_fill(ch):
        pos_v[pl.ds(ch * 16, 16)] = jnp.minimum(
            lax.iota(jnp.int32, 16) + (i0 + ch * 16), mp)

    cpos = pltpu.async_copy(pos_v, pos_hbm.at[pl.ds(row0, _MROWS_PW)], sp)

    ones16 = jnp.full((16,), _ONESW, jnp.int32)
    zero16 = jnp.zeros((16,), jnp.int32)
    wiota = lax.iota(jnp.int32, 16)
    bufs = (rb0, rb1)
    sems = (sm0, sm1)
    copies = [None, None]
    for h in range(_NHALF):
        buf = bufs[h % 2]
        if copies[h % 2] is not None:
            copies[h % 2].wait()

        @pl.loop(0, _MHALF * _NCHK, unroll=8)
        def _ones(kk):
            buf[pl.ds(kk * 16, 16)] = ones16

        @pl.loop(0, _MHALF)
        def _row(r):
            i = i0 + h * _MHALF + r
            t = jnp.maximum(i, mp)
            bk = t >> 6
            base = r * _WROW

            @pl.loop(0, bk)
            def _zeros(k):
                buf[pl.ds(base + k * 16, 16)] = zero16

            rr = t & 63
            w0 = rr >> 2
            m = rr & 3
            mixed = jnp.int32(0x01010100) << (m * 8)
            vals = jnp.where(wiota > w0, ones16,
                             jnp.where(wiota == w0, mixed, zero16))
            buf[pl.ds(base + bk * 16, 16)] = vals

        copies[h % 2] = pltpu.async_copy(
            buf,
            mask_hbm.at[pl.ds((row0 + h * _MHALF) * _WROW, _MHALF * _WROW)],
            sems[h % 2])
    copies[0].wait()
    copies[1].wait()
    cpos.wait()


_MBS = 128


def _merge_body(pad_ref, out_ref):
    out_ref[...] = pad_ref[:, :BATCH, :]


def _tc_merge(padded):
    return pl.pallas_call(
        _merge_body,
        grid=(SEQ // _MBS,),
        in_specs=[pl.BlockSpec((_MBS, 8, D_MODEL), lambda m: (m, 0, 0))],
        out_specs=pl.BlockSpec((_MBS, BATCH, D_MODEL), lambda m: (m, 0, 0)),
        out_shape=jax.ShapeDtypeStruct((SEQ, BATCH, D_MODEL), jnp.float32),
    )(padded)


def kernel(input_ids, labels, weight):
    mask_words, pos_flat = _sc_mask(input_ids)
    mask_bytes = lax.bitcast_convert_type(mask_words, jnp.int8)
    attention_mask = mask_bytes.reshape(BATCH, 1, SEQ, SEQ).astype(jnp.bool_)
    position_ids = pos_flat.reshape(BATCH, SEQ)
    ids_t = jnp.transpose(input_ids)
    k = jnp.arange(_PROWS, dtype=jnp.int32)
    oidx = ((k // BATCH) * 8 + (k % BATCH)).reshape(_NW, _PNCH, _CHUNK)
    idx = ids_t.reshape(_NW, _PNCH, _CHUNK)
    flat = _sc_gather_piece(idx, oidx, weight)
    hidden_states = _tc_merge(flat.reshape(SEQ, 8, D_MODEL))
    return (hidden_states, position_ids, attention_mask, labels)


# speedup vs baseline: 1.3938x; 1.3938x over previous
import functools

import jax
import jax.numpy as jnp
from jax import lax
from jax.experimental import pallas as pl
from jax.experimental.pallas import tpu as pltpu
from jax.experimental.pallas import tpu_sc as plsc

VOCAB = 150528
D_MODEL = 1024
BATCH = 4
SEQ = 2048
MASK_TOKEN = 150001

_INFO = plsc.get_sparse_core_info()
_NW = _INFO.num_cores * _INFO.num_subcores
_ROWS = BATCH * SEQ
_RPW = _ROWS // _NW
_CHUNK = 32
_NCH = _RPW // _CHUNK

_mesh = plsc.VectorSubcoreMesh(core_axis_name="c", subcore_axis_name="s")


_NPIECE = 1
_PROWS = _ROWS // _NPIECE
_PNCH = _PROWS // _NW // _CHUNK


def _make_sc_gather(nch):
    @functools.partial(
        pl.kernel,
        mesh=_mesh,
        out_type=jax.ShapeDtypeStruct((SEQ // _NPIECE * 8, D_MODEL),
                                      jnp.float32),
        scratch_types=[
            pltpu.VMEM((nch, _CHUNK), jnp.int32),
            pltpu.VMEM((nch, _CHUNK), jnp.int32),
            pltpu.VMEM((_CHUNK, D_MODEL), jnp.float32),
            pltpu.VMEM((_CHUNK, D_MODEL), jnp.float32),
            pltpu.SemaphoreType.DMA,
            pltpu.SemaphoreType.DMA,
            pltpu.SemaphoreType.DMA,
            pltpu.SemaphoreType.DMA,
        ],
    )
    def _sc_gather(idx_hbm, oidx_hbm, w_hbm, out_hbm, idx_v, oidx_v, buf0,
                   buf1, si0, si1, so0, so1):
        wid = lax.axis_index("s") * _INFO.num_cores + lax.axis_index("c")
        pltpu.sync_copy(idx_hbm.at[wid], idx_v)
        pltpu.sync_copy(oidx_hbm.at[wid], oidx_v)
        bufs = (buf0, buf1)
        sin = (si0, si1)
        sout = (so0, so1)
        cin = [None] * nch
        cout = [None] * nch
        cin[0] = pltpu.async_copy(w_hbm.at[idx_v.at[0]], buf0, si0)
        if nch > 1:
            cin[1] = pltpu.async_copy(w_hbm.at[idx_v.at[1]], buf1, si1)
        for c in range(nch):
            b = c % 2
            cin[c].wait()
            cout[c] = pltpu.async_copy(bufs[b], out_hbm.at[oidx_v.at[c]],
                                       sout[b])
            nxt = c + 2
            if nxt < nch:
                cout[c].wait()
                cin[nxt] = pltpu.async_copy(w_hbm.at[idx_v.at[nxt]], bufs[b],
                                            sin[b])
            else:
                cout[c].wait()

    return _sc_gather


_sc_gather_piece = _make_sc_gather(_PNCH)


_BS = 256


def _mask_body(ids_ref, amask_ref, pos_ref):
    sb = pl.program_id(1)
    ids = ids_ref[0, 0, :]
    col1 = lax.broadcasted_iota(jnp.int32, (1, SEQ), 1)
    mp = jnp.min(jnp.where(ids[None, :] == MASK_TOKEN, col1, SEQ))
    rows = sb * _BS + lax.broadcasted_iota(jnp.int32, (_BS, SEQ), 0)
    cols = lax.broadcasted_iota(jnp.int32, (_BS, SEQ), 1)
    amask_ref[0, 0] = (cols > jnp.maximum(rows, mp)).astype(jnp.int8)
    pos_ref[0] = jnp.minimum(col1, mp)


def _tc_mask(input_ids):
    amask, pos = pl.pallas_call(
        _mask_body,
        grid=(BATCH, SEQ // _BS),
        in_specs=[pl.BlockSpec((1, 1, SEQ), lambda b, sb: (b, 0, 0))],
        out_specs=[
            pl.BlockSpec((1, 1, _BS, SEQ), lambda b, sb: (b, 0, sb, 0)),
            pl.BlockSpec((1, 1, SEQ), lambda b, sb: (b, 0, 0)),
        ],
        out_shape=[
            jax.ShapeDtypeStruct((BATCH, 1, SEQ, SEQ), jnp.int8),
            jax.ShapeDtypeStruct((BATCH, 1, SEQ), jnp.int32),
        ],
    )(input_ids.reshape(BATCH, 1, SEQ))
    return amask.astype(jnp.bool_), pos.reshape(BATCH, SEQ)


_MBS = 128


def _merge_body(pad_ref, out_ref):
    out_ref[...] = pad_ref[:, 0]


def _tc_merge(padded):
    return pl.pallas_call(
        _merge_body,
        grid=(SEQ // _MBS,),
        in_specs=[pl.BlockSpec((_MBS, 1, BATCH, D_MODEL),
                               lambda m: (m, 0, 0, 0))],
        out_specs=pl.BlockSpec((_MBS, BATCH, D_MODEL), lambda m: (m, 0, 0)),
        out_shape=jax.ShapeDtypeStruct((SEQ, BATCH, D_MODEL), jnp.float32),
    )(padded)


def kernel(input_ids, labels, weight):
    attention_mask, position_ids = _tc_mask(input_ids)
    ids_t = jnp.transpose(input_ids)
    k = jnp.arange(_PROWS, dtype=jnp.int32)
    oidx = ((k // BATCH) * 8 + (k % BATCH)).reshape(_NW, _PNCH, _CHUNK)
    idx = ids_t.reshape(_NW, _PNCH, _CHUNK)
    flat = _sc_gather_piece(idx, oidx, weight)
    hidden_states = _tc_merge(flat.reshape(SEQ, 2, BATCH, D_MODEL))
    return (hidden_states, position_ids, attention_mask, labels)


# speedup vs baseline: 2.4149x; 1.7326x over previous
import functools

import jax
import jax.numpy as jnp
from jax import lax
from jax.experimental import pallas as pl
from jax.experimental.pallas import tpu as pltpu
from jax.experimental.pallas import tpu_sc as plsc

VOCAB = 150528
D_MODEL = 1024
BATCH = 4
SEQ = 2048
MASK_TOKEN = 150001

_INFO = plsc.get_sparse_core_info()
_NW = _INFO.num_cores * _INFO.num_subcores
_ROWS = BATCH * SEQ
_RPW = _ROWS // _NW
_CHUNK = 32
_NCH = _RPW // _CHUNK

_mesh = plsc.VectorSubcoreMesh(core_axis_name="c", subcore_axis_name="s")


_NPIECE = 1
_PROWS = _ROWS // _NPIECE
_PNCH = _PROWS // _NW // _CHUNK


def _make_sc_gather(nch):
    @functools.partial(
        pl.kernel,
        mesh=_mesh,
        out_type=jax.ShapeDtypeStruct((SEQ // _NPIECE * 8, D_MODEL),
                                      jnp.float32),
        scratch_types=[
            pltpu.VMEM((nch, _CHUNK), jnp.int32),
            pltpu.VMEM((nch, _CHUNK), jnp.int32),
            pltpu.VMEM((_CHUNK, D_MODEL), jnp.float32),
            pltpu.VMEM((_CHUNK, D_MODEL), jnp.float32),
            pltpu.SemaphoreType.DMA,
            pltpu.SemaphoreType.DMA,
            pltpu.SemaphoreType.DMA,
            pltpu.SemaphoreType.DMA,
        ],
    )
    def _sc_gather(idx_hbm, oidx_hbm, w_hbm, out_hbm, idx_v, oidx_v, buf0,
                   buf1, si0, si1, so0, so1):
        wid = lax.axis_index("s") * _INFO.num_cores + lax.axis_index("c")
        pltpu.sync_copy(idx_hbm.at[wid], idx_v)
        pltpu.sync_copy(oidx_hbm.at[wid], oidx_v)
        bufs = (buf0, buf1)
        sin = (si0, si1)
        sout = (so0, so1)
        cin = [None] * nch
        cout = [None] * nch
        cin[0] = pltpu.async_copy(w_hbm.at[idx_v.at[0]], buf0, si0)
        if nch > 1:
            cin[1] = pltpu.async_copy(w_hbm.at[idx_v.at[1]], buf1, si1)
        for c in range(nch):
            b = c % 2
            cin[c].wait()
            cout[c] = pltpu.async_copy(bufs[b], out_hbm.at[oidx_v.at[c]],
                                       sout[b])
            nxt = c + 2
            if nxt < nch:
                cout[c].wait()
                cin[nxt] = pltpu.async_copy(w_hbm.at[idx_v.at[nxt]], bufs[b],
                                            sin[b])
            else:
                cout[c].wait()

    return _sc_gather


_sc_gather_piece = _make_sc_gather(_PNCH)


_BS = 512


def _mask_body(ids_ref, amask_ref, pos_ref):
    sb = pl.program_id(1)
    ids = ids_ref[0, 0, :]
    col1 = lax.broadcasted_iota(jnp.int32, (1, SEQ), 1)
    mp = jnp.min(jnp.where(ids[None, :] == MASK_TOKEN, col1, SEQ))
    rows = sb * _BS + lax.broadcasted_iota(jnp.int32, (_BS, SEQ), 0)
    cols = lax.broadcasted_iota(jnp.int32, (_BS, SEQ), 1)
    amask_ref[0, 0] = (cols > jnp.maximum(rows, mp)).astype(jnp.int8)
    pos_ref[0] = jnp.minimum(col1, mp)


def _tc_mask(input_ids):
    amask, pos = pl.pallas_call(
        _mask_body,
        grid=(BATCH, SEQ // _BS),
        in_specs=[pl.BlockSpec((1, 1, SEQ), lambda b, sb: (b, 0, 0))],
        out_specs=[
            pl.BlockSpec((1, 1, _BS, SEQ), lambda b, sb: (b, 0, sb, 0)),
            pl.BlockSpec((1, 1, SEQ), lambda b, sb: (b, 0, 0)),
        ],
        out_shape=[
            jax.ShapeDtypeStruct((BATCH, 1, SEQ, SEQ), jnp.int8),
            jax.ShapeDtypeStruct((BATCH, 1, SEQ), jnp.int32),
        ],
    )(input_ids.reshape(BATCH, 1, SEQ))
    return amask.astype(jnp.bool_), pos.reshape(BATCH, SEQ)


_MBS = 256


def _merge_body(pad_ref, out_ref):
    out_ref[...] = pad_ref[:, :BATCH, :]


def _tc_merge(padded):
    return pl.pallas_call(
        _merge_body,
        grid=(SEQ // _MBS,),
        in_specs=[pl.BlockSpec((_MBS, 8, D_MODEL), lambda m: (m, 0, 0))],
        out_specs=pl.BlockSpec((_MBS, BATCH, D_MODEL), lambda m: (m, 0, 0)),
        out_shape=jax.ShapeDtypeStruct((SEQ, BATCH, D_MODEL), jnp.float32),
    )(padded)


def kernel(input_ids, labels, weight):
    attention_mask, position_ids = _tc_mask(input_ids)
    ids_t = jnp.transpose(input_ids)
    k = jnp.arange(_PROWS, dtype=jnp.int32)
    oidx = ((k // BATCH) * 8 + (k % BATCH)).reshape(_NW, _PNCH, _CHUNK)
    idx = ids_t.reshape(_NW, _PNCH, _CHUNK)
    flat = _sc_gather_piece(idx, oidx, weight)
    hidden_states = _tc_merge(flat.reshape(SEQ, 8, D_MODEL))
    return (hidden_states, position_ids, attention_mask, labels)


# speedup vs baseline: 2.4444x; 1.0122x over previous
import functools

import jax
import jax.numpy as jnp
from jax import lax
from jax.experimental import pallas as pl
from jax.experimental.pallas import tpu as pltpu
from jax.experimental.pallas import tpu_sc as plsc

VOCAB = 150528
D_MODEL = 1024
BATCH = 4
SEQ = 2048
MASK_TOKEN = 150001

_INFO = plsc.get_sparse_core_info()
_NW = _INFO.num_cores * _INFO.num_subcores
_ROWS = BATCH * SEQ
_RPW = _ROWS // _NW
_CHUNK = 32
_NCH = _RPW // _CHUNK

_mesh = plsc.VectorSubcoreMesh(core_axis_name="c", subcore_axis_name="s")


_NPIECE = 1
_PROWS = _ROWS // _NPIECE
_PNCH = _PROWS // _NW // _CHUNK


def _make_sc_gather(nch):
    @functools.partial(
        pl.kernel,
        mesh=_mesh,
        out_type=jax.ShapeDtypeStruct((SEQ // _NPIECE * 8, D_MODEL),
                                      jnp.float32),
        scratch_types=[
            pltpu.VMEM((nch, _CHUNK), jnp.int32),
            pltpu.VMEM((nch, _CHUNK), jnp.int32),
            pltpu.VMEM((_CHUNK, D_MODEL), jnp.float32),
            pltpu.VMEM((_CHUNK, D_MODEL), jnp.float32),
            pltpu.SemaphoreType.DMA,
            pltpu.SemaphoreType.DMA,
            pltpu.SemaphoreType.DMA,
            pltpu.SemaphoreType.DMA,
        ],
    )
    def _sc_gather(idx_hbm, oidx_hbm, w_hbm, out_hbm, idx_v, oidx_v, buf0,
                   buf1, si0, si1, so0, so1):
        wid = lax.axis_index("s") * _INFO.num_cores + lax.axis_index("c")
        pltpu.sync_copy(idx_hbm.at[wid], idx_v)
        pltpu.sync_copy(oidx_hbm.at[wid], oidx_v)
        bufs = (buf0, buf1)
        sin = (si0, si1)
        sout = (so0, so1)
        cin = [None] * nch
        cout = [None] * nch
        cin[0] = pltpu.async_copy(w_hbm.at[idx_v.at[0]], buf0, si0)
        if nch > 1:
            cin[1] = pltpu.async_copy(w_hbm.at[idx_v.at[1]], buf1, si1)
        for c in range(nch):
            b = c % 2
            cin[c].wait()
            cout[c] = pltpu.async_copy(bufs[b], out_hbm.at[oidx_v.at[c]],
                                       sout[b])
            nxt = c + 2
            if nxt < nch:
                cout[c].wait()
                cin[nxt] = pltpu.async_copy(w_hbm.at[idx_v.at[nxt]], bufs[b],
                                            sin[b])
            else:
                cout[c].wait()

    return _sc_gather


_sc_gather_piece = _make_sc_gather(_PNCH)


_BS = 1024


def _mask_body(ids_ref, amask_ref, pos_ref):
    sb = pl.program_id(1)
    ids = ids_ref[0, 0, :]
    col1 = lax.broadcasted_iota(jnp.int32, (1, SEQ), 1)
    mp = jnp.min(jnp.where(ids[None, :] == MASK_TOKEN, col1, SEQ))
    rows = sb * _BS + lax.broadcasted_iota(jnp.int32, (_BS, SEQ), 0)
    cols = lax.broadcasted_iota(jnp.int32, (_BS, SEQ), 1)
    amask_ref[0, 0] = (cols > jnp.maximum(rows, mp)).astype(jnp.int8)
    pos_ref[0] = jnp.minimum(col1, mp)


def _tc_mask(input_ids):
    amask, pos = pl.pallas_call(
        _mask_body,
        grid=(BATCH, SEQ // _BS),
        in_specs=[pl.BlockSpec((1, 1, SEQ), lambda b, sb: (b, 0, 0))],
        out_specs=[
            pl.BlockSpec((1, 1, _BS, SEQ), lambda b, sb: (b, 0, sb, 0)),
            pl.BlockSpec((1, 1, SEQ), lambda b, sb: (b, 0, 0)),
        ],
        out_shape=[
            jax.ShapeDtypeStruct((BATCH, 1, SEQ, SEQ), jnp.int8),
            jax.ShapeDtypeStruct((BATCH, 1, SEQ), jnp.int32),
        ],
    )(input_ids.reshape(BATCH, 1, SEQ))
    return amask.astype(jnp.bool_), pos.reshape(BATCH, SEQ)


_MBS = 512


def _merge_body(pad_ref, out_ref):
    out_ref[...] = pad_ref[:, :BATCH, :]


def _tc_merge(padded):
    return pl.pallas_call(
        _merge_body,
        grid=(SEQ // _MBS,),
        in_specs=[pl.BlockSpec((_MBS, 8, D_MODEL), lambda m: (m, 0, 0))],
        out_specs=pl.BlockSpec((_MBS, BATCH, D_MODEL), lambda m: (m, 0, 0)),
        out_shape=jax.ShapeDtypeStruct((SEQ, BATCH, D_MODEL), jnp.float32),
    )(padded)


def kernel(input_ids, labels, weight):
    attention_mask, position_ids = _tc_mask(input_ids)
    ids_t = jnp.transpose(input_ids)
    k = jnp.arange(_PROWS, dtype=jnp.int32)
    oidx = ((k // BATCH) * 8 + (k % BATCH)).reshape(_NW, _PNCH, _CHUNK)
    idx = ids_t.reshape(_NW, _PNCH, _CHUNK)
    flat = _sc_gather_piece(idx, oidx, weight)
    hidden_states = _tc_merge(flat.reshape(SEQ, 8, D_MODEL))
    return (hidden_states, position_ids, attention_mask, labels)


# speedup vs baseline: 2.4474x; 1.0012x over previous
import functools

import jax
import jax.numpy as jnp
from jax import lax
from jax.experimental import pallas as pl
from jax.experimental.pallas import tpu as pltpu
from jax.experimental.pallas import tpu_sc as plsc

VOCAB = 150528
D_MODEL = 1024
BATCH = 4
SEQ = 2048
MASK_TOKEN = 150001

_INFO = plsc.get_sparse_core_info()
_NW = _INFO.num_cores * _INFO.num_subcores
_ROWS = BATCH * SEQ
_RPW = _ROWS // _NW
_CHUNK = 32
_NCH = _RPW // _CHUNK

_mesh = plsc.VectorSubcoreMesh(core_axis_name="c", subcore_axis_name="s")


_NPIECE = 1
_PROWS = _ROWS // _NPIECE
_PNCH = _PROWS // _NW // _CHUNK


def _make_sc_gather(nch):
    @functools.partial(
        pl.kernel,
        mesh=_mesh,
        out_type=jax.ShapeDtypeStruct((SEQ // _NPIECE * 8, D_MODEL),
                                      jnp.float32),
        scratch_types=[
            pltpu.VMEM((nch, _CHUNK), jnp.int32),
            pltpu.VMEM((nch, _CHUNK), jnp.int32),
            pltpu.VMEM((_CHUNK, D_MODEL), jnp.float32),
            pltpu.VMEM((_CHUNK, D_MODEL), jnp.float32),
            pltpu.SemaphoreType.DMA,
            pltpu.SemaphoreType.DMA,
            pltpu.SemaphoreType.DMA,
            pltpu.SemaphoreType.DMA,
        ],
    )
    def _sc_gather(idx_hbm, oidx_hbm, w_hbm, out_hbm, idx_v, oidx_v, buf0,
                   buf1, si0, si1, so0, so1):
        wid = lax.axis_index("s") * _INFO.num_cores + lax.axis_index("c")
        pltpu.sync_copy(idx_hbm.at[wid], idx_v)
        pltpu.sync_copy(oidx_hbm.at[wid], oidx_v)
        bufs = (buf0, buf1)
        sin = (si0, si1)
        sout = (so0, so1)
        cin = [None] * nch
        cout = [None] * nch
        cin[0] = pltpu.async_copy(w_hbm.at[idx_v.at[0]], buf0, si0)
        if nch > 1:
            cin[1] = pltpu.async_copy(w_hbm.at[idx_v.at[1]], buf1, si1)
        for c in range(nch):
            b = c % 2
            cin[c].wait()
            cout[c] = pltpu.async_copy(bufs[b], out_hbm.at[oidx_v.at[c]],
                                       sout[b])
            nxt = c + 2
            if nxt < nch:
                cout[c].wait()
                cin[nxt] = pltpu.async_copy(w_hbm.at[idx_v.at[nxt]], bufs[b],
                                            sin[b])
            else:
                cout[c].wait()

    return _sc_gather


_sc_gather_piece = _make_sc_gather(_PNCH)


_BS = 1024


def _mask_body(ids_ref, amask_ref, pos_ref):
    sb = pl.program_id(1)
    ids = ids_ref[0, 0, :]
    col1 = lax.broadcasted_iota(jnp.int32, (1, SEQ), 1)
    mp = jnp.min(jnp.where(ids[None, :] == MASK_TOKEN, col1, SEQ))
    th = jnp.maximum(sb * _BS + lax.broadcasted_iota(jnp.int32, (_BS, 1), 0),
                     mp)
    cols = lax.broadcasted_iota(jnp.int32, (_BS, SEQ), 1)
    amask_ref[0, 0] = (cols > th).astype(jnp.int8)
    pos_ref[0] = jnp.minimum(col1, mp)


def _tc_mask(input_ids):
    amask, pos = pl.pallas_call(
        _mask_body,
        grid=(BATCH, SEQ // _BS),
        in_specs=[pl.BlockSpec((1, 1, SEQ), lambda b, sb: (b, 0, 0))],
        out_specs=[
            pl.BlockSpec((1, 1, _BS, SEQ), lambda b, sb: (b, 0, sb, 0)),
            pl.BlockSpec((1, 1, SEQ), lambda b, sb: (b, 0, 0)),
        ],
        out_shape=[
            jax.ShapeDtypeStruct((BATCH, 1, SEQ, SEQ), jnp.int8),
            jax.ShapeDtypeStruct((BATCH, 1, SEQ), jnp.int32),
        ],
    )(input_ids.reshape(BATCH, 1, SEQ))
    return amask.astype(jnp.bool_), pos.reshape(BATCH, SEQ)


_MBS = 512


def _merge_body(pad_ref, out_ref):
    out_ref[...] = pad_ref[:, :BATCH, :]


def _tc_merge(padded):
    return pl.pallas_call(
        _merge_body,
        grid=(SEQ // _MBS,),
        in_specs=[pl.BlockSpec((_MBS, 8, D_MODEL), lambda m: (m, 0, 0))],
        out_specs=pl.BlockSpec((_MBS, BATCH, D_MODEL), lambda m: (m, 0, 0)),
        out_shape=jax.ShapeDtypeStruct((SEQ, BATCH, D_MODEL), jnp.float32),
    )(padded)


def kernel(input_ids, labels, weight):
    attention_mask, position_ids = _tc_mask(input_ids)
    ids_t = jnp.transpose(input_ids)
    k = jnp.arange(_PROWS, dtype=jnp.int32)
    oidx = ((k // BATCH) * 8 + (k % BATCH)).reshape(_NW, _PNCH, _CHUNK)
    idx = ids_t.reshape(_NW, _PNCH, _CHUNK)
    flat = _sc_gather_piece(idx, oidx, weight)
    hidden_states = _tc_merge(flat.reshape(SEQ, 8, D_MODEL))
    return (hidden_states, position_ids, attention_mask, labels)


# speedup vs baseline: 2.7656x; 1.1300x over previous
import functools

import jax
import jax.numpy as jnp
from jax import lax
from jax.experimental import pallas as pl
from jax.experimental.pallas import tpu as pltpu
from jax.experimental.pallas import tpu_sc as plsc

VOCAB = 150528
D_MODEL = 1024
BATCH = 4
SEQ = 2048
MASK_TOKEN = 150001

_INFO = plsc.get_sparse_core_info()
_NW = _INFO.num_cores * _INFO.num_subcores
_ROWS = BATCH * SEQ
_RPW = _ROWS // _NW
_CHUNK = 32
_NCH = _RPW // _CHUNK

_mesh = plsc.VectorSubcoreMesh(core_axis_name="c", subcore_axis_name="s")


_NPIECE = 1
_PROWS = _ROWS // _NPIECE
_PNCH = _PROWS // _NW // _CHUNK


def _make_sc_gather(nch):
    @functools.partial(
        pl.kernel,
        mesh=_mesh,
        out_type=jax.ShapeDtypeStruct((_ROWS, D_MODEL), jnp.float32),
        scratch_types=[
            pltpu.VMEM((nch, _CHUNK), jnp.int32),
            pltpu.VMEM((_CHUNK, D_MODEL), jnp.float32),
            pltpu.VMEM((_CHUNK, D_MODEL), jnp.float32),
            pltpu.SemaphoreType.DMA,
            pltpu.SemaphoreType.DMA,
            pltpu.SemaphoreType.DMA,
            pltpu.SemaphoreType.DMA,
        ],
    )
    def _sc_gather(idx_hbm, w_hbm, out_hbm, idx_v, buf0, buf1, si0, si1, so0,
                   so1):
        wid = lax.axis_index("s") * _INFO.num_cores + lax.axis_index("c")
        pltpu.sync_copy(idx_hbm.at[wid], idx_v)
        base = wid * _RPW
        bufs = (buf0, buf1)
        sin = (si0, si1)
        sout = (so0, so1)
        cin = [None] * nch
        cout = [None] * nch
        cin[0] = pltpu.async_copy(w_hbm.at[idx_v.at[0]], buf0, si0)
        if nch > 1:
            cin[1] = pltpu.async_copy(w_hbm.at[idx_v.at[1]], buf1, si1)
        for c in range(nch):
            b = c % 2
            cin[c].wait()
            cout[c] = pltpu.async_copy(
                bufs[b], out_hbm.at[pl.ds(base + c * _CHUNK, _CHUNK)],
                sout[b])
            nxt = c + 2
            if nxt < nch:
                cout[c].wait()
                cin[nxt] = pltpu.async_copy(w_hbm.at[idx_v.at[nxt]], bufs[b],
                                            sin[b])
            else:
                cout[c].wait()

    return _sc_gather


_sc_gather_piece = _make_sc_gather(_PNCH)


_BS = 1024


def _mask_body(ids_ref, amask_ref, pos_ref):
    sb = pl.program_id(1)
    ids = ids_ref[0, 0, :]
    col1 = lax.broadcasted_iota(jnp.int32, (1, SEQ), 1)
    mp = jnp.min(jnp.where(ids[None, :] == MASK_TOKEN, col1, SEQ))
    th = jnp.maximum(sb * _BS + lax.broadcasted_iota(jnp.int32, (_BS, 1), 0),
                     mp)
    cols = lax.broadcasted_iota(jnp.int32, (_BS, SEQ), 1)
    amask_ref[0, 0] = (cols > th).astype(jnp.int8)
    pos_ref[0] = jnp.minimum(col1, mp)


def _tc_mask(input_ids):
    amask, pos = pl.pallas_call(
        _mask_body,
        grid=(BATCH, SEQ // _BS),
        in_specs=[pl.BlockSpec((1, 1, SEQ), lambda b, sb: (b, 0, 0))],
        out_specs=[
            pl.BlockSpec((1, 1, _BS, SEQ), lambda b, sb: (b, 0, sb, 0)),
            pl.BlockSpec((1, 1, SEQ), lambda b, sb: (b, 0, 0)),
        ],
        out_shape=[
            jax.ShapeDtypeStruct((BATCH, 1, SEQ, SEQ), jnp.int8),
            jax.ShapeDtypeStruct((BATCH, 1, SEQ), jnp.int32),
        ],
    )(input_ids.reshape(BATCH, 1, SEQ))
    return amask.astype(jnp.bool_), pos.reshape(BATCH, SEQ)


_MBS = 512


def _merge_body(bsd_ref, out_ref):
    out_ref[...] = jnp.transpose(bsd_ref[...], (1, 0, 2))


def _tc_merge(bsd):
    return pl.pallas_call(
        _merge_body,
        grid=(SEQ // _MBS,),
        in_specs=[pl.BlockSpec((BATCH, _MBS, D_MODEL), lambda m: (0, m, 0))],
        out_specs=pl.BlockSpec((_MBS, BATCH, D_MODEL), lambda m: (m, 0, 0)),
        out_shape=jax.ShapeDtypeStruct((SEQ, BATCH, D_MODEL), jnp.float32),
    )(bsd)


def kernel(input_ids, labels, weight):
    attention_mask, position_ids = _tc_mask(input_ids)
    idx = input_ids.reshape(_NW, _PNCH, _CHUNK)
    flat = _sc_gather_piece(idx, weight)
    hidden_states = _tc_merge(flat.reshape(BATCH, SEQ, D_MODEL))
    return (hidden_states, position_ids, attention_mask, labels)
